# baseline (speedup 1.0000x reference)
import functools

import jax
import jax.numpy as jnp
from jax import lax
from jax.experimental import pallas as pl
from jax.experimental.pallas import tpu as pltpu
from jax.experimental.pallas import tpu_sc as plsc

NC = 2
NS = 16
NW = NC * NS
BATCH = 16384
EMB = 32
BPW = BATCH // NW
NCHUNK = 4
CHUNK = BPW // NCHUNK
LANES = 16
TILE = 128
G = 4
SC_BATCH = 8192
TC_BATCH = BATCH - SC_BATCH
BPW_SC = SC_BATCH // NW
NGROUP = BPW_SC // G


def _dot_sc(inputs3, tu, tp):
    mesh = plsc.VectorSubcoreMesh(core_axis_name="c", subcore_axis_name="s")

    @functools.partial(
        pl.kernel,
        mesh=mesh,
        compiler_params=pltpu.CompilerParams(needs_layout_passes=False,
                                             disable_bounds_checks=True),
        out_type=jax.ShapeDtypeStruct((NW, LANES), jnp.float32),
        scratch_types=[
            pltpu.VMEM((BPW_SC * 2 + LANES,), jnp.int32),
            pltpu.VMEM((G, EMB, TILE), jnp.float32),
            pltpu.VMEM((G, EMB, TILE), jnp.float32),
            pltpu.VMEM((G, EMB, TILE), jnp.float32),
            pltpu.VMEM((G, EMB, TILE), jnp.float32),
            pltpu.VMEM((LANES,), jnp.float32),
            pltpu.SemaphoreType.DMA((G,)),
            pltpu.SemaphoreType.DMA((G,)),
            pltpu.SemaphoreType.DMA((G,)),
            pltpu.SemaphoreType.DMA((G,)),
        ],
    )
    def k(inputs_hbm, tu_hbm, tp_hbm, partials_hbm,
          pairs, ub0, ub1, pb0, pb1, accv, su0, su1, sp0, sp1):
        wid = lax.axis_index("s") * NC + lax.axis_index("c")

        pltpu.sync_copy(inputs_hbm.at[wid], pairs.at[pl.ds(0, BPW_SC * 2)])

        lane = jnp.arange(LANES, dtype=jnp.int32)

        def issue_slot(i, ru, rp, ub, pb, su, sp):
            l0u = pl.multiple_of((ru >> 7) * TILE, TILE)
            l0p = pl.multiple_of((rp >> 7) * TILE, TILE)
            pltpu.async_copy(
                tu_hbm.at[:, pl.ds(l0u, TILE)], ub.at[i], su.at[i])
            pltpu.async_copy(
                tp_hbm.at[:, pl.ds(l0p, TILE)], pb.at[i], sp.at[i])

        def issue(g, ub, pb, su, sp):
            v = pairs[pl.ds(g * (2 * G), LANES)]
            for i in range(G):
                issue_slot(i, v[2 * i], v[2 * i + 1], ub, pb, su, sp)

        def step(g, ub, pb, su, sp, acc):
            v = pairs[pl.ds(g * (2 * G), LANES)]
            for i in range(G):
                ru = v[2 * i]
                rp = v[2 * i + 1]
                pltpu.make_async_copy(
                    tu_hbm.at[:, pl.ds(0, TILE)], ub.at[i], su.at[i]).wait()
                pltpu.make_async_copy(
                    tp_hbm.at[:, pl.ds(0, TILE)], pb.at[i], sp.at[i]).wait()
                cu = jnp.full((LANES,), ru & (TILE - 1), jnp.int32)
                cp = jnp.full((LANES,), rp & (TILE - 1), jnp.int32)
                u0 = plsc.load_gather(ub.at[i], [lane, cu])
                u1 = plsc.load_gather(ub.at[i], [lane + LANES, cu])
                p0 = plsc.load_gather(pb.at[i], [lane, cp])
                p1 = plsc.load_gather(pb.at[i], [lane + LANES, cp])
                acc = acc + u0 * p0 + u1 * p1

                @pl.when(g + 2 < NGROUP)
                def _():
                    v2 = pairs[pl.ds((g + 2) * (2 * G), LANES)]
                    issue_slot(i, v2[2 * i], v2[2 * i + 1], ub, pb, su, sp)

            return acc

        issue(0, ub0, pb0, su0, sp0)
        issue(1, ub1, pb1, su1, sp1)

        def body(g2, acc):
            g = g2 * 2
            acc = step(g, ub0, pb0, su0, sp0, acc)
            acc = step(g + 1, ub1, pb1, su1, sp1, acc)
            return acc

        acc = lax.fori_loop(0, NGROUP // 2, body,
                            jnp.zeros((LANES,), jnp.float32))
        accv[...] = acc
        pltpu.sync_copy(accv, partials_hbm.at[wid])

    return k(inputs3, tu, tp)


def _dot_tc(tu, tp, ublk, pblk, ucol, pcol):
    def body(ublk_ref, pblk_ref, ucol_ref, pcol_ref, ub_ref, pb_ref, out_ref):
        i = pl.program_id(0)
        cu = ucol_ref[i]
        cp = pcol_ref[i]
        iota = lax.broadcasted_iota(jnp.int32, (EMB, TILE), 1)
        ohu = jnp.where(iota == cu, 1.0, 0.0)
        ohp = jnp.where(iota == cp, 1.0, 0.0)
        a = jnp.sum(ub_ref[...] * ohu, axis=1)
        b = jnp.sum(pb_ref[...] * ohp, axis=1)
        contrib = jnp.sum(a * b)

        @pl.when(i == 0)
        def _():
            out_ref[...] = jnp.zeros_like(out_ref)

        out_ref[...] += contrib

    grid_spec = pltpu.PrefetchScalarGridSpec(
        num_scalar_prefetch=4,
        grid=(TC_BATCH,),
        in_specs=[
            pl.BlockSpec((EMB, TILE), lambda i, ub, pb, uc, pc: (0, ub[i])),
            pl.BlockSpec((EMB, TILE), lambda i, ub, pb, uc, pc: (0, pb[i])),
        ],
        out_specs=pl.BlockSpec((8, TILE), lambda i, ub, pb, uc, pc: (0, 0)),
    )
    return pl.pallas_call(
        body,
        grid_spec=grid_spec,
        out_shape=jax.ShapeDtypeStruct((8, TILE), jnp.float32),
    )(ublk, pblk, ucol, pcol, tu, tp)


def _post_sc(inputs3, ub, pb, partials, tcpart):
    mesh = plsc.VectorSubcoreMesh(core_axis_name="c", subcore_axis_name="s")

    @functools.partial(
        pl.kernel,
        mesh=mesh,
        compiler_params=pltpu.CompilerParams(use_tc_tiling_on_sc=False,
                                             needs_layout_passes=False),
        out_type=jax.ShapeDtypeStruct((BATCH,), jnp.float32),
        scratch_types=[
            pltpu.VMEM((BPW, 2), jnp.int32),
            pltpu.VMEM((NCHUNK, CHUNK), jnp.int32),
            pltpu.VMEM((NCHUNK, CHUNK), jnp.int32),
            pltpu.VMEM((BPW,), jnp.float32),
            pltpu.VMEM((BPW,), jnp.float32),
            pltpu.VMEM((BPW,), jnp.float32),
            pltpu.VMEM((NW, LANES), jnp.float32),
            pltpu.VMEM((8, TILE), jnp.float32),
            pltpu.SemaphoreType.DMA,
        ],
    )
    def k(inputs_hbm, ub_hbm, pb_hbm, partials_hbm, tcpart_hbm, out_hbm,
          pairs, idx_u, idx_p, bu, bp, bsum, pv, tcv, sem):
        wid = lax.axis_index("s") * NC + lax.axis_index("c")
        base = wid * BPW

        pltpu.sync_copy(inputs_hbm.at[wid], pairs)
        pltpu.sync_copy(partials_hbm, pv)
        pltpu.sync_copy(tcpart_hbm, tcv)

        lane = jnp.arange(LANES, dtype=jnp.int32)
        zeros = jnp.zeros((LANES,), jnp.int32)
        ones = zeros + 1
        for j in range(NCHUNK):
            for t in range(CHUNK // LANES):
                rows = j * CHUNK + t * LANES + lane
                s = pl.ds(t * LANES, LANES)
                idx_u[j, s] = plsc.load_gather(pairs, [rows, zeros])
                idx_p[j, s] = plsc.load_gather(pairs, [rows, ones])

        copies = []
        for j in range(NCHUNK):
            sl = pl.ds(j * CHUNK, CHUNK)
            copies.append(pltpu.async_copy(
                ub_hbm.at[idx_u.at[j]], bu.at[sl], sem))
            copies.append(pltpu.async_copy(
                pb_hbm.at[idx_p.at[j]], bp.at[sl], sem))
        for cp in copies:
            cp.wait()

        acc = jnp.zeros((LANES,), jnp.float32)
        for w in range(NW):
            acc = acc + pv[w, pl.ds(0, LANES)]
        tvec = tcv[0, pl.ds(0, LANES)]
        stot = lax.reduce_sum_p.bind(acc, axes=(0,)) + tvec[0]
        svec = jnp.full((LANES,), stot, jnp.float32)

        def bias_body(i, carry):
            s = pl.ds(pl.multiple_of(i * LANES, LANES), LANES)
            x = bu[s] + bp[s] + svec
            bsum[s] = 1.0 / (1.0 + jnp.exp(-x))
            return carry
        lax.fori_loop(0, BPW // LANES, bias_body, 0)
        pltpu.sync_copy(bsum, out_hbm.at[pl.ds(base, BPW)])

    return k(inputs3, ub, pb, partials, tcpart)


def kernel(inputs, user_table, user_bias_table, prodi_table, prodi_bias_table):
    inputs3 = inputs.reshape(NW, BPW, 2)
    tu = user_table.T
    tp = prodi_table.T
    ub = user_bias_table.reshape(-1)
    pb = prodi_bias_table.reshape(-1)

    sc_pairs = inputs[:SC_BATCH].reshape(NW, BPW_SC * 2)
    u_idx_tc = inputs[SC_BATCH:, 0]
    p_idx_tc = inputs[SC_BATCH:, 1]

    partials = _dot_sc(sc_pairs, tu, tp)
    tcpart = _dot_tc(tu, tp,
                     u_idx_tc >> 7, p_idx_tc >> 7,
                     u_idx_tc & (TILE - 1), p_idx_tc & (TILE - 1))
    out = _post_sc(inputs3, ub, pb, partials, tcpart)
    return out.reshape(BATCH, 1)


# speedup vs baseline: 17.6180x; 17.6180x over previous
import functools

import jax
import jax.numpy as jnp
from jax import lax
from jax.experimental import pallas as pl
from jax.experimental.pallas import tpu as pltpu
from jax.experimental.pallas import tpu_sc as plsc

NC = 2
NS = 16
NW = NC * NS
BATCH = 16384
EMB = 32
BPW = BATCH // NW
NCHUNK = 4
CHUNK = BPW // NCHUNK
LANES = 16
TILE = 128
G = 4
NGROUP = BPW // G


def _dot_sc(inputs3, tu, tp):
    mesh = plsc.VectorSubcoreMesh(core_axis_name="c", subcore_axis_name="s")

    @functools.partial(
        pl.kernel,
        mesh=mesh,
        compiler_params=pltpu.CompilerParams(needs_layout_passes=False,
                                             disable_bounds_checks=True),
        out_type=jax.ShapeDtypeStruct((NW, LANES), jnp.float32),
        scratch_types=[
            pltpu.VMEM((BPW * 2 + LANES,), jnp.int32),
            pltpu.VMEM((G, EMB, TILE), jnp.float32),
            pltpu.VMEM((G, EMB, TILE), jnp.float32),
            pltpu.VMEM((G, EMB, TILE), jnp.float32),
            pltpu.VMEM((G, EMB, TILE), jnp.float32),
            pltpu.VMEM((LANES,), jnp.float32),
            pltpu.SemaphoreType.DMA((G,)),
            pltpu.SemaphoreType.DMA((G,)),
            pltpu.SemaphoreType.DMA((G,)),
            pltpu.SemaphoreType.DMA((G,)),
        ],
    )
    def k(inputs_hbm, tu_hbm, tp_hbm, partials_hbm,
          pairs, ub0, ub1, pb0, pb1, accv, su0, su1, sp0, sp1):
        wid = lax.axis_index("s") * NC + lax.axis_index("c")

        pltpu.sync_copy(inputs_hbm.at[wid], pairs.at[pl.ds(0, BPW * 2)])

        lane = jnp.arange(LANES, dtype=jnp.int32)

        def issue_slot(i, ru, rp, ub, pb, su, sp):
            l0u = pl.multiple_of((ru >> 7) * TILE, TILE)
            l0p = pl.multiple_of((rp >> 7) * TILE, TILE)
            for f in range(EMB // 8):
                sf = pl.ds(f * 8, 8)
                pltpu.async_copy(
                    tu_hbm.at[sf, pl.ds(l0u, TILE)], ub.at[i, sf], su.at[i])
                pltpu.async_copy(
                    tp_hbm.at[sf, pl.ds(l0p, TILE)], pb.at[i, sf], sp.at[i])

        def issue(g, ub, pb, su, sp):
            v = pairs[pl.ds(g * (2 * G), LANES)]
            for i in range(G):
                issue_slot(i, v[2 * i], v[2 * i + 1], ub, pb, su, sp)

        def step(g, ub, pb, su, sp, acc):
            v = pairs[pl.ds(g * (2 * G), LANES)]
            for i in range(G):
                ru = v[2 * i]
                rp = v[2 * i + 1]
                for f in range(EMB // 8):
                    sf = pl.ds(f * 8, 8)
                    pltpu.make_async_copy(
                        tu_hbm.at[sf, pl.ds(0, TILE)],
                        ub.at[i, sf], su.at[i]).wait()
                    pltpu.make_async_copy(
                        tp_hbm.at[sf, pl.ds(0, TILE)],
                        pb.at[i, sf], sp.at[i]).wait()
                cu = jnp.full((LANES,), ru & (TILE - 1), jnp.int32)
                cp = jnp.full((LANES,), rp & (TILE - 1), jnp.int32)
                u0 = plsc.load_gather(ub.at[i], [lane, cu])
                u1 = plsc.load_gather(ub.at[i], [lane + LANES, cu])
                p0 = plsc.load_gather(pb.at[i], [lane, cp])
                p1 = plsc.load_gather(pb.at[i], [lane + LANES, cp])
                acc = acc + u0 * p0 + u1 * p1

                @pl.when(g + 2 < NGROUP)
                def _():
                    v2 = pairs[pl.ds((g + 2) * (2 * G), LANES)]
                    issue_slot(i, v2[2 * i], v2[2 * i + 1], ub, pb, su, sp)

            return acc

        issue(0, ub0, pb0, su0, sp0)
        issue(1, ub1, pb1, su1, sp1)

        def body(g2, acc):
            g = g2 * 2
            acc = step(g, ub0, pb0, su0, sp0, acc)
            acc = step(g + 1, ub1, pb1, su1, sp1, acc)
            return acc

        acc = lax.fori_loop(0, NGROUP // 2, body,
                            jnp.zeros((LANES,), jnp.float32))
        accv[...] = acc
        pltpu.sync_copy(accv, partials_hbm.at[wid])

    return k(inputs3, tu, tp)


def _post_sc(inputs3, ub, pb, partials):
    mesh = plsc.VectorSubcoreMesh(core_axis_name="c", subcore_axis_name="s")

    @functools.partial(
        pl.kernel,
        mesh=mesh,
        compiler_params=pltpu.CompilerParams(use_tc_tiling_on_sc=False,
                                             needs_layout_passes=False),
        out_type=jax.ShapeDtypeStruct((BATCH,), jnp.float32),
        scratch_types=[
            pltpu.VMEM((BPW, 2), jnp.int32),
            pltpu.VMEM((NCHUNK, CHUNK), jnp.int32),
            pltpu.VMEM((NCHUNK, CHUNK), jnp.int32),
            pltpu.VMEM((BPW,), jnp.float32),
            pltpu.VMEM((BPW,), jnp.float32),
            pltpu.VMEM((BPW,), jnp.float32),
            pltpu.VMEM((NW, LANES), jnp.float32),
            pltpu.SemaphoreType.DMA,
        ],
    )
    def k(inputs_hbm, ub_hbm, pb_hbm, partials_hbm, out_hbm,
          pairs, idx_u, idx_p, bu, bp, bsum, pv, sem):
        wid = lax.axis_index("s") * NC + lax.axis_index("c")
        base = wid * BPW

        pltpu.sync_copy(inputs_hbm.at[wid], pairs)
        pltpu.sync_copy(partials_hbm, pv)

        lane = jnp.arange(LANES, dtype=jnp.int32)
        zeros = jnp.zeros((LANES,), jnp.int32)
        ones = zeros + 1
        for j in range(NCHUNK):
            for t in range(CHUNK // LANES):
                rows = j * CHUNK + t * LANES + lane
                s = pl.ds(t * LANES, LANES)
                idx_u[j, s] = plsc.load_gather(pairs, [rows, zeros])
                idx_p[j, s] = plsc.load_gather(pairs, [rows, ones])

        copies = []
        for j in range(NCHUNK):
            sl = pl.ds(j * CHUNK, CHUNK)
            copies.append(pltpu.async_copy(
                ub_hbm.at[idx_u.at[j]], bu.at[sl], sem))
            copies.append(pltpu.async_copy(
                pb_hbm.at[idx_p.at[j]], bp.at[sl], sem))
        for cp in copies:
            cp.wait()

        acc = jnp.zeros((LANES,), jnp.float32)
        for w in range(NW):
            acc = acc + pv[w, pl.ds(0, LANES)]
        stot = lax.reduce_sum_p.bind(acc, axes=(0,))
        svec = jnp.full((LANES,), stot, jnp.float32)

        def bias_body(i, carry):
            s = pl.ds(pl.multiple_of(i * LANES, LANES), LANES)
            x = bu[s] + bp[s] + svec
            bsum[s] = 1.0 / (1.0 + jnp.exp(-x))
            return carry
        lax.fori_loop(0, BPW // LANES, bias_body, 0)
        pltpu.sync_copy(bsum, out_hbm.at[pl.ds(base, BPW)])

    return k(inputs3, ub, pb, partials)


def kernel(inputs, user_table, user_bias_table, prodi_table, prodi_bias_table):
    inputs3 = inputs.reshape(NW, BPW, 2)
    inputs2 = inputs.reshape(NW, BPW * 2)
    tu = user_table.T
    tp = prodi_table.T
    ub = user_bias_table.reshape(-1)
    pb = prodi_bias_table.reshape(-1)

    partials = _dot_sc(inputs2, tu, tp)
    out = _post_sc(inputs3, ub, pb, partials)
    return out.reshape(BATCH, 1)
